# baseline (speedup 1.0000x reference)
import functools

import jax
import jax.numpy as jnp
from jax import lax
from jax.experimental import pallas as pl
from jax.experimental.pallas import tpu as pltpu
from jax.experimental.pallas import tpu_sc as plsc

_B = 16384
_D = 64
_L = 16
_NC = 2
_NS = 16
_NW = _NC * _NS
_BPW = _B // _NW
_GCHUNK = 128

_mesh = plsc.VectorSubcoreMesh(core_axis_name="c", subcore_axis_name="s")


@functools.partial(
    pl.kernel,
    mesh=_mesh,
    compiler_params=pltpu.CompilerParams(use_tc_tiling_on_sc=False),
    out_type=jax.ShapeDtypeStruct((_NW, _L), jnp.float32),
    scratch_types=[
        pltpu.VMEM((_BPW,), jnp.int32),
        pltpu.VMEM((_BPW, _D), jnp.float32),
        pltpu.VMEM((_BPW, _D), jnp.float32),
        pltpu.VMEM((_L,), jnp.float32),
        pltpu.SemaphoreType.DMA,
        pltpu.SemaphoreType.DMA,
    ],
)
def _center_loss_partials(pred_hbm, labels_hbm, center_hbm, out_hbm,
                          idx_v, rows_v, pred_v, acc_v, gsem, psem):
    wid = lax.axis_index("s") * _NC + lax.axis_index("c")
    base = wid * _BPW

    pltpu.sync_copy(labels_hbm.at[pl.ds(base, _BPW)], idx_v)

    pred_cp = pltpu.async_copy(pred_hbm.at[pl.ds(base, _BPW), :], pred_v, psem)
    gathers = []
    for j in range(_BPW // _GCHUNK):
        gathers.append(pltpu.async_copy(
            center_hbm.at[idx_v.at[pl.ds(j * _GCHUNK, _GCHUNK)]],
            rows_v.at[pl.ds(j * _GCHUNK, _GCHUNK), :],
            gsem))
    pred_cp.wait()
    for g in gathers:
        g.wait()

    def body(i, accs):
        a0, a1, a2, a3 = accs
        d0 = pred_v[i, pl.ds(0, _L)] - rows_v[i, pl.ds(0, _L)]
        d1 = pred_v[i, pl.ds(_L, _L)] - rows_v[i, pl.ds(_L, _L)]
        d2 = pred_v[i, pl.ds(2 * _L, _L)] - rows_v[i, pl.ds(2 * _L, _L)]
        d3 = pred_v[i, pl.ds(3 * _L, _L)] - rows_v[i, pl.ds(3 * _L, _L)]
        return (a0 + d0 * d0, a1 + d1 * d1, a2 + d2 * d2, a3 + d3 * d3)

    z = jnp.zeros((_L,), jnp.float32)
    a0, a1, a2, a3 = lax.fori_loop(0, _BPW, body, (z, z, z, z))
    acc_v[...] = ((a0 + a1) + (a2 + a3)) * (0.5 / _B)
    pltpu.sync_copy(acc_v, out_hbm.at[wid])


def kernel(pred, labels, center):
    partials = _center_loss_partials(pred, labels, center)
    return jnp.sum(partials)


# speedup vs baseline: 2.3248x; 2.3248x over previous
import functools

import jax
import jax.numpy as jnp
from jax import lax
from jax.experimental import pallas as pl
from jax.experimental.pallas import tpu as pltpu
from jax.experimental.pallas import tpu_sc as plsc

_B = 16384
_D = 64
_NCLS = 100000
_L = 16
_NC = 2
_NS = 16
_NW = _NC * _NS
_FPW = _D // _NW
_PC = 4096
_NCHUNK = _B // _PC
_GPC = _PC // _L

_mesh = plsc.VectorSubcoreMesh(core_axis_name="c", subcore_axis_name="s")


@functools.partial(
    pl.kernel,
    mesh=_mesh,
    compiler_params=pltpu.CompilerParams(needs_layout_passes=False),
    out_type=jax.ShapeDtypeStruct((_NW * _L,), jnp.float32),
    scratch_types=[
        pltpu.VMEM((_B,), jnp.int32),
        pltpu.VMEM((_NCLS,), jnp.float32),
        pltpu.VMEM((_PC,), jnp.float32),
        pltpu.VMEM((_PC,), jnp.float32),
        pltpu.VMEM((_L,), jnp.float32),
        pltpu.SemaphoreType.DMA,
        pltpu.SemaphoreType.DMA,
        pltpu.SemaphoreType.DMA,
    ],
)
def _center_loss_partials(pred_hbm, labels_hbm, center_hbm, out_hbm,
                          lbl_v, crow_v, pb0, pb1, acc_v,
                          csem, psem0, psem1):
    wid = lax.axis_index("s") * _NC + lax.axis_index("c")

    pltpu.sync_copy(labels_hbm, lbl_v)

    pbufs = (pb0, pb1)
    psems = (psem0, psem1)

    acc = jnp.zeros((_L,), jnp.float32)
    for phase in range(_FPW):
        f = wid * _FPW + phase
        crow_cp = pltpu.async_copy(center_hbm.at[f], crow_v, csem)
        pcopies = [pltpu.async_copy(
            pred_hbm.at[f, pl.ds(0, _PC)], pb0, psem0)]
        crow_cp.wait()

        for c in range(_NCHUNK):
            if c + 1 < _NCHUNK:
                pcopies.append(pltpu.async_copy(
                    pred_hbm.at[f, pl.ds((c + 1) * _PC, _PC)],
                    pbufs[(c + 1) % 2], psems[(c + 1) % 2]))
            pcopies[c].wait()
            pbuf = pbufs[c % 2]

            def group_body(g, a, c=c, pbuf=pbuf):
                lvec = lbl_v[pl.ds(c * _PC + g * _L, _L)]
                cvec = plsc.load_gather(crow_v, [lvec])
                pvec = pbuf[pl.ds(g * _L, _L)]
                d = pvec - cvec
                return a + d * d

            acc = lax.fori_loop(0, _GPC, group_body, acc, unroll=4)

    acc_v[...] = acc * (0.5 / _B)
    pltpu.sync_copy(acc_v, out_hbm.at[pl.ds(wid * _L, _L)])


def kernel(pred, labels, center):
    partials = _center_loss_partials(pred.T, labels, center.T)
    return jnp.sum(partials)


# speedup vs baseline: 2.3986x; 1.0318x over previous
import functools

import jax
import jax.numpy as jnp
from jax import lax
from jax.experimental import pallas as pl
from jax.experimental.pallas import tpu as pltpu
from jax.experimental.pallas import tpu_sc as plsc

_B = 16384
_D = 64
_NCLS = 100000
_L = 16
_NC = 2
_NS = 16
_NW = _NC * _NS
_FPW = _D // _NW
_PC = 4096
_NCHUNK = _B // _PC
_GPC = _PC // _L

_mesh = plsc.VectorSubcoreMesh(core_axis_name="c", subcore_axis_name="s")


@functools.partial(
    pl.kernel,
    mesh=_mesh,
    compiler_params=pltpu.CompilerParams(needs_layout_passes=False),
    out_type=jax.ShapeDtypeStruct((_NW * _L,), jnp.float32),
    scratch_types=[
        pltpu.VMEM((_B,), jnp.int32),
        pltpu.VMEM((_NCLS,), jnp.float32),
        pltpu.VMEM((_PC,), jnp.float32),
        pltpu.VMEM((_PC,), jnp.float32),
        pltpu.VMEM((_L,), jnp.float32),
        pltpu.SemaphoreType.DMA,
        pltpu.SemaphoreType.DMA,
        pltpu.SemaphoreType.DMA,
        pltpu.SemaphoreType.DMA,
    ],
)
def _center_loss_partials(pred_hbm, labels_hbm, center_hbm, out_hbm,
                          lbl_v, crow_v, pb0, pb1, acc_v,
                          csem, psem0, psem1, lsem):
    wid = lax.axis_index("s") * _NC + lax.axis_index("c")

    lbl_cp = pltpu.async_copy(labels_hbm, lbl_v, lsem)

    pbufs = (pb0, pb1)
    psems = (psem0, psem1)

    acc = jnp.zeros((_L,), jnp.float32)
    for phase in range(_FPW):
        f = wid * _FPW + phase
        crow_cp = pltpu.async_copy(center_hbm.at[f], crow_v, csem)
        pcopies = [pltpu.async_copy(
            pred_hbm.at[f, pl.ds(0, _PC)], pb0, psem0)]
        if phase == 0:
            lbl_cp.wait()
        crow_cp.wait()

        for c in range(_NCHUNK):
            if c + 1 < _NCHUNK:
                pcopies.append(pltpu.async_copy(
                    pred_hbm.at[f, pl.ds((c + 1) * _PC, _PC)],
                    pbufs[(c + 1) % 2], psems[(c + 1) % 2]))
            pcopies[c].wait()
            pbuf = pbufs[c % 2]

            def group_body(g, a, c=c, pbuf=pbuf):
                lvec = lbl_v[pl.ds(c * _PC + g * _L, _L)]
                cvec = plsc.load_gather(crow_v, [lvec])
                pvec = pbuf[pl.ds(g * _L, _L)]
                d = pvec - cvec
                return a + d * d

            acc = lax.fori_loop(0, _GPC, group_body, acc, unroll=8)

    acc_v[...] = acc * (0.5 / _B)
    pltpu.sync_copy(acc_v, out_hbm.at[pl.ds(wid * _L, _L)])


def kernel(pred, labels, center):
    partials = _center_loss_partials(pred.T, labels, center.T)
    return jnp.sum(partials)


# speedup vs baseline: 2.5827x; 1.0767x over previous
import functools

import jax
import jax.numpy as jnp
from jax import lax
from jax.experimental import pallas as pl
from jax.experimental.pallas import tpu as pltpu
from jax.experimental.pallas import tpu_sc as plsc

_B = 16384
_D = 64
_NCLS = 100000
_L = 16
_NC = 2
_NS = 16
_NW = _NC * _NS
_FPW = _D // _NW
_PC = 4096
_NCHUNK = _B // _PC
_GPC = _PC // _L

_mesh = plsc.VectorSubcoreMesh(core_axis_name="c", subcore_axis_name="s")


@functools.partial(
    pl.kernel,
    mesh=_mesh,
    compiler_params=pltpu.CompilerParams(needs_layout_passes=False),
    out_type=jax.ShapeDtypeStruct((_NW * _L,), jnp.float32),
    scratch_types=[
        pltpu.VMEM_SHARED((_B,), jnp.int32),
        pltpu.VMEM((_B,), jnp.int32),
        pltpu.VMEM((_NCLS,), jnp.float32),
        pltpu.VMEM((_PC,), jnp.float32),
        pltpu.VMEM((_PC,), jnp.float32),
        pltpu.VMEM((_L,), jnp.float32),
        pltpu.SemaphoreType.DMA,
        pltpu.SemaphoreType.DMA,
        pltpu.SemaphoreType.DMA,
        pltpu.SemaphoreType.DMA,
    ],
)
def _center_loss_partials(pred_hbm, labels_hbm, center_hbm, out_hbm,
                          lbl_sh, lbl_v, crow_v, pb0, pb1, acc_v,
                          csem, psem0, psem1, lsem):
    sid = lax.axis_index("s")
    wid = sid * _NC + lax.axis_index("c")

    crow_cp0 = pltpu.async_copy(center_hbm.at[wid * _FPW], crow_v, csem)
    pcp0 = pltpu.async_copy(pred_hbm.at[wid * _FPW, pl.ds(0, _PC)], pb0, psem0)

    @pl.when(sid == 0)
    def _stage_labels():
        pltpu.sync_copy(labels_hbm, lbl_sh)

    plsc.subcore_barrier()
    lbl_cp = pltpu.async_copy(lbl_sh, lbl_v, lsem)

    pbufs = (pb0, pb1)
    psems = (psem0, psem1)

    acc = jnp.zeros((_L,), jnp.float32)
    for phase in range(_FPW):
        f = wid * _FPW + phase
        if phase == 0:
            crow_cp = crow_cp0
            pcopies = [pcp0]
            lbl_cp.wait()
        else:
            crow_cp = pltpu.async_copy(center_hbm.at[f], crow_v, csem)
            pcopies = [pltpu.async_copy(
                pred_hbm.at[f, pl.ds(0, _PC)], pb0, psem0)]
        crow_cp.wait()

        for c in range(_NCHUNK):
            if c + 1 < _NCHUNK:
                pcopies.append(pltpu.async_copy(
                    pred_hbm.at[f, pl.ds((c + 1) * _PC, _PC)],
                    pbufs[(c + 1) % 2], psems[(c + 1) % 2]))
            pcopies[c].wait()
            pbuf = pbufs[c % 2]

            def group_body(g, a, c=c, pbuf=pbuf):
                lvec = lbl_v[pl.ds(c * _PC + g * _L, _L)]
                cvec = plsc.load_gather(crow_v, [lvec])
                pvec = pbuf[pl.ds(g * _L, _L)]
                d = pvec - cvec
                return a + d * d

            acc = lax.fori_loop(0, _GPC, group_body, acc, unroll=8)

    acc_v[...] = acc * (0.5 / _B)
    pltpu.sync_copy(acc_v, out_hbm.at[pl.ds(wid * _L, _L)])


def kernel(pred, labels, center):
    partials = _center_loss_partials(pred.T, labels, center.T)
    return jnp.sum(partials)


# speedup vs baseline: 2.5944x; 1.0045x over previous
import functools

import jax
import jax.numpy as jnp
from jax import lax
from jax.experimental import pallas as pl
from jax.experimental.pallas import tpu as pltpu
from jax.experimental.pallas import tpu_sc as plsc

_B = 16384
_D = 64
_NCLS = 100000
_L = 16
_NC = 2
_NS = 16
_NW = _NC * _NS
_FPW = _D // _NW
_PC = 4096
_NCHUNK = _B // _PC
_GPC = _PC // _L

_mesh = plsc.VectorSubcoreMesh(core_axis_name="c", subcore_axis_name="s")


@functools.partial(
    pl.kernel,
    mesh=_mesh,
    compiler_params=pltpu.CompilerParams(needs_layout_passes=False),
    out_type=jax.ShapeDtypeStruct((_NW * _L,), jnp.float32),
    scratch_types=[
        pltpu.VMEM_SHARED((_B,), jnp.int32),
        pltpu.VMEM((_B,), jnp.int32),
        pltpu.VMEM((_NCLS,), jnp.float32),
        pltpu.VMEM((_PC,), jnp.float32),
        pltpu.VMEM((_PC,), jnp.float32),
        pltpu.VMEM((_L,), jnp.float32),
        pltpu.SemaphoreType.DMA,
        pltpu.SemaphoreType.DMA,
        pltpu.SemaphoreType.DMA,
        pltpu.SemaphoreType.DMA,
    ],
)
def _center_loss_partials(pred_hbm, labels_hbm, center_hbm, out_hbm,
                          lbl_sh, lbl_v, crow_v, pb0, pb1, acc_v,
                          csem, psem0, psem1, lsem):
    sid = lax.axis_index("s")
    wid = sid * _NC + lax.axis_index("c")

    pltpu.async_copy(center_hbm.at[wid * _FPW], crow_v, csem)

    @pl.when(sid == 0)
    def _stage_labels():
        pltpu.sync_copy(labels_hbm, lbl_sh)

    plsc.subcore_barrier()
    pltpu.sync_copy(lbl_sh, lbl_v)

    pbufs = (pb0, pb1)
    psems = (psem0, psem1)

    def phase_body(phase, acc):
        f = wid * _FPW + phase
        pcopies = [pltpu.async_copy(
            pred_hbm.at[f, pl.ds(0, _PC)], pb0, psem0)]
        pltpu.make_async_copy(center_hbm.at[f], crow_v, csem).wait()

        for c in range(_NCHUNK):
            if c + 1 < _NCHUNK:
                pcopies.append(pltpu.async_copy(
                    pred_hbm.at[f, pl.ds((c + 1) * _PC, _PC)],
                    pbufs[(c + 1) % 2], psems[(c + 1) % 2]))
            pcopies[c].wait()
            pbuf = pbufs[c % 2]

            def group_body(g, a, c=c, pbuf=pbuf):
                lvec = lbl_v[pl.ds(c * _PC + g * _L, _L)]
                cvec = plsc.load_gather(crow_v, [lvec])
                pvec = pbuf[pl.ds(g * _L, _L)]
                d = pvec - cvec
                return a + d * d

            acc = lax.fori_loop(0, _GPC, group_body, acc, unroll=8)

        @pl.when(phase + 1 < _FPW)
        def _fire_next_crow():
            pltpu.async_copy(center_hbm.at[f + 1], crow_v, csem)

        return acc

    acc = lax.fori_loop(0, _FPW, phase_body, jnp.zeros((_L,), jnp.float32))

    acc_v[...] = acc * (0.5 / _B)
    pltpu.sync_copy(acc_v, out_hbm.at[pl.ds(wid * _L, _L)])


def kernel(pred, labels, center):
    partials = _center_loss_partials(pred.T, labels, center.T)
    return jnp.sum(partials)
